# baseline (speedup 1.0000x reference)
import functools

import jax
import jax.numpy as jnp
from jax.experimental import pallas as pl
from jax.experimental.pallas import tpu as pltpu

_C = 1024
_K = 8
_D = 256

_BB = 1024
_BC = 2048


def _fused_kernel(z_ref, y_ref, p_ref, pos_ref, neg_ref):
    j = pl.program_id(1)

    zt = z_ref[...]
    zn = zt * jax.lax.rsqrt(
        jnp.maximum(jnp.sum(zt * zt, axis=1, keepdims=True), 1e-24))
    pt = p_ref[...]
    pn = pt * jax.lax.rsqrt(
        jnp.maximum(jnp.sum(pt * pt, axis=1, keepdims=True), 1e-24))

    sim = jax.lax.dot_general(
        zn.astype(jnp.bfloat16), pn.astype(jnp.bfloat16),
        dimension_numbers=(((1,), (1,)), ((), ())),
        preferred_element_type=jnp.float32)

    col = jax.lax.broadcasted_iota(jnp.int32, (_BB, _BC), 1)
    cls = (j * _BC + col) >> 3
    same = cls == y_ref[...][:, None]

    ninf = jnp.float32(-jnp.inf)
    pos_c = jnp.max(jnp.where(same, sim, ninf), axis=1)
    neg_c = jnp.max(jnp.where(same, ninf, sim), axis=1)

    @pl.when(j == 0)
    def _init():
        pos_ref[...] = pos_c
        neg_ref[...] = neg_c

    @pl.when(j != 0)
    def _acc():
        pos_ref[...] = jnp.maximum(pos_ref[...], pos_c)
        neg_ref[...] = jnp.maximum(neg_ref[...], neg_c)


@functools.partial(jax.jit, static_argnums=())
def kernel(z, y, protos):
    B = z.shape[0]
    P = protos.reshape(_C * _K, _D)
    grid = (B // _BB, (_C * _K) // _BC)
    pos, neg = pl.pallas_call(
        _fused_kernel,
        grid=grid,
        in_specs=[
            pl.BlockSpec((_BB, _D), lambda i, j: (i, 0)),
            pl.BlockSpec((_BB,), lambda i, j: (i,)),
            pl.BlockSpec((_BC, _D), lambda i, j: (j, 0)),
        ],
        out_specs=[
            pl.BlockSpec((_BB,), lambda i, j: (i,)),
            pl.BlockSpec((_BB,), lambda i, j: (i,)),
        ],
        out_shape=[
            jax.ShapeDtypeStruct((B,), jnp.float32),
            jax.ShapeDtypeStruct((B,), jnp.float32),
        ],
        compiler_params=pltpu.CompilerParams(
            dimension_semantics=("parallel", "arbitrary")),
    )(z, y, P)
    return (pos, neg)


# speedup vs baseline: 1.1078x; 1.1078x over previous
import jax
import jax.numpy as jnp
from jax.experimental import pallas as pl
from jax.experimental.pallas import tpu as pltpu

_C = 1024
_K = 8
_D = 256

_BB = 1024
_BC = 2048


def _normalize_kernel(z_ref, p_ref, zn_ref, pn_ref):
    zt = z_ref[...]
    zn_ref[...] = (zt * jax.lax.rsqrt(
        jnp.maximum(jnp.sum(zt * zt, axis=1, keepdims=True), 1e-24))
    ).astype(jnp.bfloat16)
    pt = p_ref[...]
    pn_ref[...] = (pt * jax.lax.rsqrt(
        jnp.maximum(jnp.sum(pt * pt, axis=1, keepdims=True), 1e-24))
    ).astype(jnp.bfloat16)


def _fused_kernel(zn_ref, y_ref, pn_ref, pos_ref, neg_ref):
    j = pl.program_id(1)

    simt = jax.lax.dot_general(
        pn_ref[...], zn_ref[...],
        dimension_numbers=(((1,), (1,)), ((), ())),
        preferred_element_type=jnp.float32)

    m = jnp.max(simt.reshape(_BC // _K, _K, _BB), axis=1)

    cls = j * (_BC // _K) + jax.lax.broadcasted_iota(
        jnp.int32, (_BC // _K, _BB), 0)
    same = cls == y_ref[...][None, :]

    ninf = jnp.float32(-jnp.inf)
    pos_c = jnp.max(jnp.where(same, m, ninf), axis=0)
    neg_c = jnp.max(jnp.where(same, ninf, m), axis=0)

    @pl.when(j == 0)
    def _init():
        pos_ref[...] = pos_c
        neg_ref[...] = neg_c

    @pl.when(j != 0)
    def _acc():
        pos_ref[...] = jnp.maximum(pos_ref[...], pos_c)
        neg_ref[...] = jnp.maximum(neg_ref[...], neg_c)


def kernel(z, y, protos):
    B = z.shape[0]
    P = protos.reshape(_C * _K, _D)

    zn, pn = pl.pallas_call(
        _normalize_kernel,
        out_shape=[
            jax.ShapeDtypeStruct((B, _D), jnp.bfloat16),
            jax.ShapeDtypeStruct((_C * _K, _D), jnp.bfloat16),
        ],
    )(z, P)

    grid = (B // _BB, (_C * _K) // _BC)
    pos, neg = pl.pallas_call(
        _fused_kernel,
        grid=grid,
        in_specs=[
            pl.BlockSpec((_BB, _D), lambda i, j: (i, 0)),
            pl.BlockSpec((_BB,), lambda i, j: (i,)),
            pl.BlockSpec((_BC, _D), lambda i, j: (j, 0)),
        ],
        out_specs=[
            pl.BlockSpec((_BB,), lambda i, j: (i,)),
            pl.BlockSpec((_BB,), lambda i, j: (i,)),
        ],
        out_shape=[
            jax.ShapeDtypeStruct((B,), jnp.float32),
            jax.ShapeDtypeStruct((B,), jnp.float32),
        ],
        compiler_params=pltpu.CompilerParams(
            dimension_semantics=("parallel", "arbitrary")),
    )(zn, y, pn)
    return (pos, neg)
